# baseline (speedup 1.0000x reference)
import jax
import jax.numpy as jnp
from jax import lax
from jax.experimental import pallas as pl
from jax.experimental.pallas import tpu as pltpu
from jax.experimental.pallas import tpu_sc as plsc


def _sc_body(thr_hbm, img_hbm, out_hbm, thr_v, buf_r, buf_g, buf_b, row_v,
             colacc, ymax, red):
    c = lax.axis_index("c")
    s = lax.axis_index("s")
    wid = s * 2 + c

    @pl.when(wid < 16)
    def _():
        img = wid
        pltpu.sync_copy(thr_hbm, thr_v)

        for cc in range(32):
            colacc[pl.ds(cc * 16, 16)] = jnp.zeros((16,), jnp.int32)
        ymax[...] = jnp.full((16,), -1, jnp.int32)

        def chunk_body(chunk, _):
            row0 = chunk * 16
            pltpu.sync_copy(img_hbm.at[img, 0, pl.ds(row0, 16), :], buf_r)
            pltpu.sync_copy(img_hbm.at[img, 1, pl.ds(row0, 16), :], buf_g)
            pltpu.sync_copy(img_hbm.at[img, 2, pl.ds(row0, 16), :], buf_b)

            for r in range(16):
                def col_body(ccd, _, r=r):
                    sl = pl.ds(ccd * 16, 16)
                    gray = (buf_r[r, sl] * 0.299 + buf_g[r, sl] * 0.587 +
                            buf_b[r, sl] * 0.114)
                    blk = gray < thr_v[...]
                    colacc[sl] = jnp.maximum(
                        colacc[sl], jnp.where(blk, jnp.int32(1), jnp.int32(0)))
                    yv = jnp.where(blk, row0 + r, jnp.int32(-1))
                    ymax[...] = jnp.maximum(ymax[...], yv)
                    return 0

                lax.fori_loop(0, 32, col_body, 0)

            pltpu.sync_copy(buf_r, out_hbm.at[img, 0, pl.ds(row0, 16), :])
            pltpu.sync_copy(buf_g, out_hbm.at[img, 1, pl.ds(row0, 16), :])
            pltpu.sync_copy(buf_b, out_hbm.at[img, 2, pl.ds(row0, 16), :])
            return 0

        lax.fori_loop(0, 32, chunk_body, 0)

        ymv = ymax[...]
        y1 = ymv[0]
        for k in range(1, 16):
            y1 = jnp.maximum(y1, ymv[k])

        x0v = jnp.full((16,), 512, jnp.int32)
        x1v = jnp.full((16,), -1, jnp.int32)
        for cc in range(32):
            sl = pl.ds(cc * 16, 16)
            idx = lax.iota(jnp.int32, 16) + cc * 16
            cav = colacc[sl] > 0
            x0v = jnp.minimum(x0v, jnp.where(cav, idx, jnp.int32(512)))
            x1v = jnp.maximum(x1v, jnp.where(cav, idx, jnp.int32(-1)))
        x0 = x0v[0]
        x1 = x1v[0]
        for k in range(1, 16):
            x0 = jnp.minimum(x0, x0v[k])
            x1 = jnp.maximum(x1, x1v[k])

        for j in range(3):
            rowi = y1 - 2 + j

            @pl.when((rowi >= 0) & (rowi <= y1))
            def _():
                for ch in range(3):
                    pltpu.sync_copy(img_hbm.at[img, ch, rowi, :], row_v)

                    def fix_body(ccd, _):
                        sl = pl.ds(ccd * 16, 16)
                        idx = lax.iota(jnp.int32, 16) + ccd * 16
                        keep = jnp.where((idx >= x0) & (idx < x1), 0.0, 1.0)
                        row_v[sl] = row_v[sl] * keep
                        return 0

                    lax.fori_loop(0, 32, fix_body, 0)
                    pltpu.sync_copy(row_v, out_hbm.at[img, ch, rowi, :])


def kernel(img_tensor, threshold):
    B, C, H, W = img_tensor.shape
    thr_arr = jnp.full((16,), threshold, jnp.float32)
    mesh = plsc.VectorSubcoreMesh(core_axis_name="c", subcore_axis_name="s")
    sc_kernel = pl.kernel(
        _sc_body,
        out_type=jax.ShapeDtypeStruct((B, C, H, W), img_tensor.dtype),
        mesh=mesh,
        scratch_types=[
            pltpu.VMEM((16,), jnp.float32),
            pltpu.VMEM((16, 512), jnp.float32),
            pltpu.VMEM((16, 512), jnp.float32),
            pltpu.VMEM((16, 512), jnp.float32),
            pltpu.VMEM((512,), jnp.float32),
            pltpu.VMEM((512,), jnp.int32),
            pltpu.VMEM((16,), jnp.int32),
            pltpu.VMEM((16,), jnp.int32),
        ],
    )
    return sc_kernel(thr_arr, img_tensor)


# speedup vs baseline: 12.7397x; 12.7397x over previous
import jax
import jax.numpy as jnp
from jax.experimental import pallas as pl
from jax.experimental.pallas import tpu as pltpu

_BB = 8


def _underline_kernel(thr_ref, in_ref, out_ref, sem):
    thr = thr_ref[0, 0]
    H, W = in_ref.shape[2], in_ref.shape[3]
    b = pl.program_id(0)

    for i in range(_BB):
        img = in_ref[i]
        gray = img[0] * 0.299 + img[1] * 0.587 + img[2] * 0.114
        black = gray < thr

        ys2d = jax.lax.broadcasted_iota(jnp.int32, (H, W), 0)
        y1 = jnp.max(jnp.where(black, ys2d, jnp.int32(-1)))

        col_any = jnp.any(black, axis=0, keepdims=True)
        xs = jax.lax.broadcasted_iota(jnp.int32, (1, W), 1)
        x0 = jnp.min(jnp.where(col_any, xs, jnp.int32(W)))
        x1 = jnp.max(jnp.where(col_any, xs, jnp.int32(-1)))

        start = pl.multiple_of(jnp.clip(((y1 - 2) // 8) * 8, 0, H - 16), 8)
        wys = start + jax.lax.broadcasted_iota(jnp.int32, (16, 1), 0)
        row_in = ((wys <= y1) & (wys >= y1 - 2)).astype(jnp.float32)
        col_in = ((xs >= x0) & (xs < x1)).astype(jnp.float32)
        keep = 1.0 - row_in * col_in
        win = in_ref[i, :, pl.ds(start, 16), :]
        in_ref[i, :, pl.ds(start, 16), :] = win * keep[None, :, :]

        pltpu.make_async_copy(in_ref.at[i], out_ref.at[b * _BB + i], sem).start()

    for i in range(_BB):
        pltpu.make_async_copy(in_ref.at[i], out_ref.at[b * _BB + i], sem).wait()


def kernel(img_tensor, threshold):
    B, C, H, W = img_tensor.shape
    thr = jnp.asarray(threshold, jnp.float32).reshape(1, 1)
    return pl.pallas_call(
        _underline_kernel,
        grid=(B // _BB,),
        in_specs=[
            pl.BlockSpec(memory_space=pltpu.SMEM),
            pl.BlockSpec((_BB, C, H, W), lambda b: (b, 0, 0, 0)),
        ],
        out_specs=pl.BlockSpec(memory_space=pltpu.MemorySpace.HBM),
        out_shape=jax.ShapeDtypeStruct((B, C, H, W), img_tensor.dtype),
        scratch_shapes=[pltpu.SemaphoreType.DMA],
        compiler_params=pltpu.CompilerParams(
            dimension_semantics=("arbitrary",),
        ),
    )(thr, img_tensor)
